# baseline (speedup 1.0000x reference)
import functools

import jax
import jax.numpy as jnp
from jax import lax
from jax.experimental import pallas as pl
from jax.experimental.pallas import tpu as pltpu
from jax.experimental.pallas import tpu_sc as plsc

_TBL = 1001
_BATCH = 16384
_NC, _NS, _L = 2, 16, 16
_NTAB = 6
_NQ = 4
_Q = _BATCH // _NQ
_CHUNKS = _Q // _L


def _body(t_hbm, b_hbm, ab_hbm, sab_hbm, somab_hbm, sra_hbm, sig_hbm,
          out_hbm, idx_v, tab_v, out_v, sem_i, sem_t, sem_o):
    wid = lax.axis_index("s") * _NC + lax.axis_index("c")

    @pl.when(wid < _NTAB * _NQ)
    def _():
        j = wid // _NQ
        q = wid % _NQ
        base = q * _Q

        half = _Q // 2
        idx_cps = [
            pltpu.make_async_copy(
                t_hbm.at[pl.ds(base + h * half, half)],
                idx_v.at[pl.ds(h * half, half)], sem_i)
            for h in range(2)
        ]
        for cp in idx_cps:
            cp.start()
        srcs = (b_hbm, sig_hbm, ab_hbm, sab_hbm, somab_hbm, sra_hbm)
        for jj, src in enumerate(srcs):
            @pl.when(j == jj)
            def _():
                pltpu.make_async_copy(src, tab_v, sem_t).start()
        pltpu.make_async_copy(srcs[0], tab_v, sem_t).wait()

        out_cps = [
            pltpu.make_async_copy(
                out_v.at[pl.ds(h * half, half)],
                out_hbm.at[pl.ds(j * _BATCH + base + h * half, half)], sem_o)
            for h in range(2)
        ]
        for h in range(2):
            idx_cps[h].wait()

            @plsc.parallel_loop(h * _CHUNKS // 2, (h + 1) * _CHUNKS // 2,
                                unroll=2)
            def _(c):
                idx = idx_v[pl.ds(c * _L, _L)]
                out_v[pl.ds(c * _L, _L)] = plsc.load_gather(tab_v, [idx])

            out_cps[h].start()
        for cp in out_cps:
            cp.wait()


@jax.jit
def kernel(t, betas, alphas_bar, sqrt_alphas_bar, sqrt_one_minus_alphas_bar,
           sqrt_recip_alphas, sigmas):
    run = functools.partial(
        pl.kernel,
        mesh=plsc.VectorSubcoreMesh(core_axis_name="c", subcore_axis_name="s"),
        compiler_params=pltpu.CompilerParams(
            needs_layout_passes=False,
            disable_bounds_checks=True,
            skip_device_barrier=True,
        ),
        out_type=jax.ShapeDtypeStruct((_NTAB * _BATCH,), jnp.float32),
        scratch_types=[
            pltpu.VMEM((_Q,), jnp.int32),
            pltpu.VMEM((_TBL,), jnp.float32),
            pltpu.VMEM((_Q,), jnp.float32),
            pltpu.SemaphoreType.DMA,
            pltpu.SemaphoreType.DMA,
            pltpu.SemaphoreType.DMA,
        ],
    )(_body)
    out = run(t.astype(jnp.int32), betas, alphas_bar, sqrt_alphas_bar,
              sqrt_one_minus_alphas_bar, sqrt_recip_alphas, sigmas)
    return out.reshape(_NTAB, _BATCH, 1, 1, 1)


# speedup vs baseline: 1.0162x; 1.0162x over previous
import functools

import jax
import jax.numpy as jnp
from jax import lax
from jax.experimental import pallas as pl
from jax.experimental.pallas import tpu as pltpu
from jax.experimental.pallas import tpu_sc as plsc

_TBL = 1001
_BATCH = 16384
_NC, _NS, _L = 2, 16, 16
_NTAB = 6
_NQ = 4
_Q = _BATCH // _NQ
_CHUNKS = _Q // _L


def _body(t_hbm, b_hbm, ab_hbm, sab_hbm, somab_hbm, sra_hbm, sig_hbm,
          out_hbm, idx_v, tab_v, out_v, sem_i, sem_t, sem_o):
    wid = lax.axis_index("s") * _NC + lax.axis_index("c")

    @pl.when(wid < _NTAB * _NQ)
    def _():
        j = wid // _NQ
        q = wid % _NQ
        base = q * _Q

        half = _Q // 2
        idx_cps = [
            pltpu.make_async_copy(
                t_hbm.at[pl.ds(base + h * half, half)],
                idx_v.at[pl.ds(h * half, half)], sem_i)
            for h in range(2)
        ]
        for cp in idx_cps:
            cp.start()
        srcs = (b_hbm, sig_hbm, ab_hbm, sab_hbm, somab_hbm, sra_hbm)
        for jj, src in enumerate(srcs):
            @pl.when(j == jj)
            def _():
                pltpu.make_async_copy(src, tab_v, sem_t).start()
        pltpu.make_async_copy(srcs[0], tab_v, sem_t).wait()

        out_cps = [
            pltpu.make_async_copy(
                out_v.at[pl.ds(h * half, half)],
                out_hbm.at[pl.ds(j * _BATCH + base + h * half, half)], sem_o)
            for h in range(2)
        ]
        for h in range(2):
            idx_cps[h].wait()

            @plsc.parallel_loop(h * _CHUNKS // 2, (h + 1) * _CHUNKS // 2,
                                unroll=4)
            def _(c):
                idx = idx_v[pl.ds(c * _L, _L)]
                out_v[pl.ds(c * _L, _L)] = plsc.load_gather(tab_v, [idx])

            out_cps[h].start()
        for cp in out_cps:
            cp.wait()


@jax.jit
def kernel(t, betas, alphas_bar, sqrt_alphas_bar, sqrt_one_minus_alphas_bar,
           sqrt_recip_alphas, sigmas):
    run = functools.partial(
        pl.kernel,
        mesh=plsc.VectorSubcoreMesh(core_axis_name="c", subcore_axis_name="s"),
        compiler_params=pltpu.CompilerParams(
            needs_layout_passes=False,
            disable_bounds_checks=True,
            skip_device_barrier=True,
        ),
        out_type=jax.ShapeDtypeStruct((_NTAB * _BATCH,), jnp.float32),
        scratch_types=[
            pltpu.VMEM((_Q,), jnp.int32),
            pltpu.VMEM((_TBL,), jnp.float32),
            pltpu.VMEM((_Q,), jnp.float32),
            pltpu.SemaphoreType.DMA,
            pltpu.SemaphoreType.DMA,
            pltpu.SemaphoreType.DMA,
        ],
    )(_body)
    out = run(t.astype(jnp.int32), betas, alphas_bar, sqrt_alphas_bar,
              sqrt_one_minus_alphas_bar, sqrt_recip_alphas, sigmas)
    return out.reshape(_NTAB, _BATCH, 1, 1, 1)
